# Initial kernel scaffold; baseline (speedup 1.0000x reference)
import jax
import jax.numpy as jnp
from jax.experimental import pallas as pl


def kernel(x, edge_index, W, b):
    raise NotImplementedError("write your pallas kernel here")


# speedup vs baseline: 12.5011x; 12.5011x over previous
import functools

import jax
import jax.numpy as jnp
from jax import lax
from jax.experimental import pallas as pl
from jax.experimental.pallas import tpu as pltpu
from jax.experimental.pallas import tpu_sc as plsc

_LANES = 16
_CH = 4


def _make_sc_gather_max(rows_total, C, K):
    NC, NS = 2, 16
    NW = NC * NS
    assert rows_total % (NW * _CH) == 0
    rows_per_w = rows_total // NW
    num_chunks = rows_per_w // _CH
    mesh = plsc.VectorSubcoreMesh(core_axis_name="c", subcore_axis_name="s")

    def body(xrows_hbm, ej_hbm, ei_hbm, out_hbm,
             idxj_v, idxi_v, xj_v, xi_v, out_v, semj, semi):
        wid = lax.axis_index("s") * NC + lax.axis_index("c")
        row0 = wid * rows_per_w

        @pl.loop(0, num_chunks)
        def _chunk(t):
            rbase = row0 + t * _CH
            fbase = rbase * K
            pltpu.sync_copy(ej_hbm.at[pl.ds(fbase, _CH * K)], idxj_v)
            pltpu.sync_copy(ei_hbm.at[pl.ds(fbase, _CH * K)], idxi_v)
            cpj = pltpu.async_copy(xrows_hbm.at[idxj_v], xj_v, semj)
            cpi = pltpu.async_copy(xrows_hbm.at[idxi_v], xi_v, semi)
            cpj.wait()
            cpi.wait()
            for r in range(_CH):
                for cs in range(C // _LANES):
                    sl = pl.ds(cs * _LANES, _LANES)
                    m = xj_v[r * K, sl] - xi_v[r * K, sl]
                    for k in range(1, K):
                        m = jnp.maximum(m, xj_v[r * K + k, sl] - xi_v[r * K + k, sl])
                    out_v[r, sl] = m
            pltpu.sync_copy(out_v, out_hbm.at[pl.ds(rbase, _CH)])

    return pl.kernel(
        body,
        out_type=jax.ShapeDtypeStruct((rows_total, C), jnp.float32),
        mesh=mesh,
        scratch_types=[
            pltpu.VMEM((_CH * K,), jnp.int32),
            pltpu.VMEM((_CH * K,), jnp.int32),
            pltpu.VMEM((_CH * K, C), jnp.float32),
            pltpu.VMEM((_CH * K, C), jnp.float32),
            pltpu.VMEM((_CH, C), jnp.float32),
            pltpu.SemaphoreType.DMA,
            pltpu.SemaphoreType.DMA,
        ],
    )


def _mm_body(x_ref, xm_ref, we_ref, wo_ref, b_ref, o_ref):
    acc = jnp.dot(we_ref[...], x_ref[0], preferred_element_type=jnp.float32)
    acc = acc + lax.dot_general(
        wo_ref[...], xm_ref[0], (((1,), (1,)), ((), ())),
        preferred_element_type=jnp.float32)
    acc = acc + b_ref[...]
    o_ref[0] = jnp.maximum(acc, 0.0)


def _tc_conv(x_flat, xmax3, W_e, W_o, bias_col, NT=512):
    B, C, N = x_flat.shape
    OUT_C = W_e.shape[0]
    return pl.pallas_call(
        _mm_body,
        grid=(B, N // NT),
        in_specs=[
            pl.BlockSpec((1, C, NT), lambda b, t: (b, 0, t)),
            pl.BlockSpec((1, NT, C), lambda b, t: (b, t, 0)),
            pl.BlockSpec((OUT_C, C), lambda b, t: (0, 0)),
            pl.BlockSpec((OUT_C, C), lambda b, t: (0, 0)),
            pl.BlockSpec((OUT_C, 1), lambda b, t: (0, 0)),
        ],
        out_specs=pl.BlockSpec((1, OUT_C, NT), lambda b, t: (b, 0, t)),
        out_shape=jax.ShapeDtypeStruct((B, OUT_C, N), jnp.float32),
    )(x_flat, xmax3, W_e, W_o, bias_col)


def kernel(x, edge_index, W, b):
    B, C, D, H, Wsp = x.shape
    n = D * H * Wsp
    K = edge_index.shape[-1]
    R = B * n

    x_flat = x.reshape(B, C, n)
    x_rows = x_flat.transpose(0, 2, 1).reshape(R, C)

    off = (jnp.arange(B, dtype=jnp.int32) * n)[None, :, None, None]
    e = edge_index + off
    ej = e[0].reshape(R * K)
    ei = e[1].reshape(R * K)

    xmax = _make_sc_gather_max(R, C, K)(x_rows, ej, ei)

    W_e = W[:, 0::2]
    W_o = W[:, 1::2]
    out = _tc_conv(x_flat, xmax.reshape(B, n, C), W_e, W_o, b.reshape(-1, 1))
    return out.reshape(B, W.shape[0], D, H, Wsp)


# speedup vs baseline: 45.4742x; 3.6376x over previous
import functools

import jax
import jax.numpy as jnp
from jax import lax
from jax.experimental import pallas as pl
from jax.experimental.pallas import tpu as pltpu
from jax.experimental.pallas import tpu_sc as plsc

_LANES = 16
_CH = 8
_NC, _NS = 2, 16
_NW = _NC * _NS


def _make_sc_gather_max(rows_total, C, K):
    assert rows_total % (_NW * _CH) == 0
    rows_per_w = rows_total // _NW
    num_chunks = rows_per_w // _CH
    assert num_chunks % 2 == 0
    mesh = plsc.VectorSubcoreMesh(core_axis_name="c", subcore_axis_name="s")

    def body(xrows_hbm, ej_hbm, ei_hbm, out_hbm,
             idxj, idxi, xj0, xj1, xi0, xi1, out_v, sj0, sj1, si0, si1):
        wid = lax.axis_index("s") * _NC + lax.axis_index("c")
        row0 = wid * rows_per_w

        pltpu.sync_copy(ej_hbm.at[wid], idxj)
        pltpu.sync_copy(ei_hbm.at[wid], idxi)

        bufs = ((xj0, xi0, sj0, si0), (xj1, xi1, sj1, si1))

        def start(t, bi):
            xj, xi, sj, si = bufs[bi]
            pltpu.async_copy(xrows_hbm.at[idxj.at[t]], xj, sj)
            pltpu.async_copy(xrows_hbm.at[idxi.at[t]], xi, si)

        def wait_buf(bi):
            xj, xi, sj, si = bufs[bi]
            pltpu.make_async_copy(xrows_hbm.at[pl.ds(0, _CH * K)], xj, sj).wait()
            pltpu.make_async_copy(xrows_hbm.at[pl.ds(0, _CH * K)], xi, si).wait()

        def compute(t, bi):
            xj, xi, _, _ = bufs[bi]

            @pl.loop(0, _CH)
            def _row(r):
                base = r * K
                for cs in range(C // _LANES):
                    sl = pl.ds(cs * _LANES, _LANES)
                    m = xj[base, sl] - xi[base, sl]
                    for k in range(1, K):
                        m = jnp.maximum(m, xj[base + k, sl] - xi[base + k, sl])
                    out_v[r, sl] = m

            pltpu.sync_copy(out_v, out_hbm.at[pl.ds(row0 + t * _CH, _CH)])

        start(0, 0)

        @pl.loop(0, num_chunks, step=2)
        def _pipe(t):
            start(t + 1, 1)
            wait_buf(0)
            compute(t, 0)
            t2 = lax.select(t + 2 < num_chunks, t + 2, 0)
            start(t2, 0)
            wait_buf(1)
            compute(t + 1, 1)

        wait_buf(0)

    return pl.kernel(
        body,
        out_type=jax.ShapeDtypeStruct((rows_total, C), jnp.float32),
        mesh=mesh,
        scratch_types=[
            pltpu.VMEM((num_chunks, _CH * K), jnp.int32),
            pltpu.VMEM((num_chunks, _CH * K), jnp.int32),
            pltpu.VMEM((_CH * K, C), jnp.float32),
            pltpu.VMEM((_CH * K, C), jnp.float32),
            pltpu.VMEM((_CH * K, C), jnp.float32),
            pltpu.VMEM((_CH * K, C), jnp.float32),
            pltpu.VMEM((_CH, C), jnp.float32),
            pltpu.SemaphoreType.DMA,
            pltpu.SemaphoreType.DMA,
            pltpu.SemaphoreType.DMA,
            pltpu.SemaphoreType.DMA,
        ],
    )


def _mm_body(x_ref, xm_ref, we_ref, wo_ref, b_ref, o_ref):
    acc = jnp.dot(we_ref[...], x_ref[0], preferred_element_type=jnp.float32)
    acc = acc + lax.dot_general(
        wo_ref[...], xm_ref[0], (((1,), (1,)), ((), ())),
        preferred_element_type=jnp.float32)
    acc = acc + b_ref[...]
    o_ref[0] = jnp.maximum(acc, 0.0)


def _tc_conv(x_flat, xmax3, W_e, W_o, bias_col, NT=512):
    B, C, N = x_flat.shape
    OUT_C = W_e.shape[0]
    return pl.pallas_call(
        _mm_body,
        grid=(B, N // NT),
        in_specs=[
            pl.BlockSpec((1, C, NT), lambda b, t: (b, 0, t)),
            pl.BlockSpec((1, NT, C), lambda b, t: (b, t, 0)),
            pl.BlockSpec((OUT_C, C), lambda b, t: (0, 0)),
            pl.BlockSpec((OUT_C, C), lambda b, t: (0, 0)),
            pl.BlockSpec((OUT_C, 1), lambda b, t: (0, 0)),
        ],
        out_specs=pl.BlockSpec((1, OUT_C, NT), lambda b, t: (b, 0, t)),
        out_shape=jax.ShapeDtypeStruct((B, OUT_C, N), jnp.float32),
    )(x_flat, xmax3, W_e, W_o, bias_col)


def kernel(x, edge_index, W, b):
    B, C, D, H, Wsp = x.shape
    n = D * H * Wsp
    K = edge_index.shape[-1]
    R = B * n

    x_flat = x.reshape(B, C, n)
    x_rows = x_flat.transpose(0, 2, 1).reshape(R, C)

    off = (jnp.arange(B, dtype=jnp.int32) * n)[None, :, None, None]
    e = edge_index + off
    rows_per_w = R // _NW
    num_chunks = rows_per_w // _CH
    ej = e[0].reshape(_NW, num_chunks, _CH * K)
    ei = e[1].reshape(_NW, num_chunks, _CH * K)

    xmax = _make_sc_gather_max(R, C, K)(x_rows, ej, ei)

    W_e = W[:, 0::2]
    W_o = W[:, 1::2]
    out = _tc_conv(x_flat, xmax.reshape(B, n, C), W_e, W_o, b.reshape(-1, 1))
    return out.reshape(B, W.shape[0], D, H, Wsp)
